# Initial kernel scaffold; baseline (speedup 1.0000x reference)
import jax
import jax.numpy as jnp
from jax.experimental import pallas as pl


def kernel(states_action, states_graph_ids, states_first, states_second, sates_num_edges, Wm, bm, Wx, Wh, b_gru, Wr1, br1, Wr2, br2, Wr3, br3):
    raise NotImplementedError("write your pallas kernel here")


# speedup vs baseline: 4.9227x; 4.9227x over previous
import functools

import jax
import jax.numpy as jnp
from jax import lax
from jax.experimental import pallas as pl
from jax.experimental.pallas import tpu as pltpu
from jax.experimental.pallas import tpu_sc as plsc

N = 10000
D = 128
E = 320000
G = 64
T = 4
RB = 2000
K = 80
NT = 32
EPT = E // NT
ACC_N = 10240
SUB_ROWS = ACC_N // 16
STG = 128

_SCALE = 1.0507009873554805
_ALPHA = 1.6732632423543772
_SA = _SCALE * _ALPHA


def _selu(x):
    return jnp.where(x > 0, _SCALE * x, _SA * jnp.exp(x) - _SA)


def _sc_edge_body(a_hbm, b_hbm, first_hbm, second_hbm, out_hbm,
             acc_sh, idx1_v, idx2_v, rows_a, rows_b, stage_v, sem_a, sem_b):
    c = lax.axis_index("c")
    s = lax.axis_index("s")
    tile = c * 16 + s
    zero = jnp.zeros((16,), jnp.float32)

    @pl.loop(0, STG)
    def _zero_stage(r):
        for k in range(D // 16):
            stage_v[r, pl.ds(k * 16, 16)] = zero

    for b in range(SUB_ROWS // STG):
        pltpu.sync_copy(stage_v, acc_sh.at[pl.ds(s * SUB_ROWS + b * STG, STG)])
    plsc.subcore_barrier()

    @pl.loop(0, EPT // K)
    def _chunk(j):
        base = tile * EPT + j * K
        pltpu.sync_copy(first_hbm.at[pl.ds(base, K)], idx1_v)
        pltpu.sync_copy(second_hbm.at[pl.ds(base, K)], idx2_v)
        cp_a = pltpu.async_copy(a_hbm.at[idx1_v], rows_a, sem_a)
        cp_b = pltpu.async_copy(b_hbm.at[idx2_v], rows_b, sem_b)
        cp_a.wait()
        cp_b.wait()

        @pl.loop(0, K)
        def _row(r):
            for k in range(D // 16):
                x = rows_a[r, pl.ds(k * 16, 16)] + rows_b[r, pl.ds(k * 16, 16)]
                rows_a[r, pl.ds(k * 16, 16)] = _selu(x)

        pltpu.sync_copy(rows_a, acc_sh.at[idx2_v], add=True)

    plsc.subcore_barrier()
    for b in range(SUB_ROWS // STG):
        row0 = s * SUB_ROWS + b * STG
        pltpu.sync_copy(acc_sh.at[pl.ds(row0, STG)], stage_v)
        pltpu.sync_copy(stage_v, out_hbm.at[c].at[pl.ds(row0, STG)])


@functools.cache
def _get_sc_edge():
    mesh = plsc.VectorSubcoreMesh(core_axis_name="c", subcore_axis_name="s")
    return pl.kernel(
        _sc_edge_body,
        out_type=jax.ShapeDtypeStruct((2, ACC_N, D), jnp.float32),
        mesh=mesh,
        scratch_types=[
            pltpu.VMEM_SHARED((ACC_N, D), jnp.float32),
            pltpu.VMEM((K,), jnp.int32),
            pltpu.VMEM((K,), jnp.int32),
            pltpu.VMEM((K, D), jnp.float32),
            pltpu.VMEM((K, D), jnp.float32),
            pltpu.VMEM((STG, D), jnp.float32),
            pltpu.SemaphoreType.DMA,
            pltpu.SemaphoreType.DMA,
        ],
    )


def _tc_prep_body(ls_ref, wm_ref, bm_ref, a_ref, b_ref):
    x = ls_ref[...]
    wm = wm_ref[...]
    a_ref[...] = jnp.dot(x, wm[:D, :], preferred_element_type=jnp.float32)
    b_ref[...] = jnp.dot(x, wm[D:, :], preferred_element_type=jnp.float32) + bm_ref[...]


def _tc_prep(ls, wm, bm2):
    return pl.pallas_call(
        _tc_prep_body,
        grid=(N // RB,),
        in_specs=[
            pl.BlockSpec((RB, D), lambda i: (i, 0)),
            pl.BlockSpec((2 * D, D), lambda i: (0, 0)),
            pl.BlockSpec((1, D), lambda i: (0, 0)),
        ],
        out_specs=[pl.BlockSpec((RB, D), lambda i: (i, 0))] * 2,
        out_shape=[jax.ShapeDtypeStruct((N, D), jnp.float32)] * 2,
    )(ls, wm, bm2)


def _tc_gru_body(p_ref, ls_ref, wx_ref, wh_ref, bg_ref, wm_ref, bm_ref,
                 out_ls, out_a, out_b):
    x = p_ref[0] + p_ref[1]
    h = ls_ref[...]
    mx = jnp.dot(x, wx_ref[...], preferred_element_type=jnp.float32) + bg_ref[0:1, :]
    mh = jnp.dot(h, wh_ref[...], preferred_element_type=jnp.float32) + bg_ref[1:2, :]
    z = jax.nn.sigmoid(mx[:, :D] + mh[:, :D])
    r = jax.nn.sigmoid(mx[:, D:2 * D] + mh[:, D:2 * D])
    hh = jnp.tanh(mx[:, 2 * D:] + r * mh[:, 2 * D:])
    ls_new = z * h + (1.0 - z) * hh
    out_ls[...] = ls_new
    wm = wm_ref[...]
    out_a[...] = jnp.dot(ls_new, wm[:D, :], preferred_element_type=jnp.float32)
    out_b[...] = jnp.dot(ls_new, wm[D:, :], preferred_element_type=jnp.float32) + bm_ref[...]


def _tc_gru(parts, ls, wx, wh, bg, wm, bm2):
    return pl.pallas_call(
        _tc_gru_body,
        grid=(N // RB,),
        in_specs=[
            pl.BlockSpec((2, RB, D), lambda i: (0, i, 0)),
            pl.BlockSpec((RB, D), lambda i: (i, 0)),
            pl.BlockSpec((D, 3 * D), lambda i: (0, 0)),
            pl.BlockSpec((D, 3 * D), lambda i: (0, 0)),
            pl.BlockSpec((2, 3 * D), lambda i: (0, 0)),
            pl.BlockSpec((2 * D, D), lambda i: (0, 0)),
            pl.BlockSpec((1, D), lambda i: (0, 0)),
        ],
        out_specs=[pl.BlockSpec((RB, D), lambda i: (i, 0))] * 3,
        out_shape=[jax.ShapeDtypeStruct((N, D), jnp.float32)] * 3,
    )(parts, ls, wx, wh, bg, wm, bm2)


def _tc_readout_body(gid_ref, ls_ref, w1_ref, b1_ref, w2_ref, b2_ref,
                     w3_ref, b3_ref, out_ref, acc_ref):
    i = pl.program_id(0)
    g = gid_ref[0, 0, :]
    onehot = (lax.broadcasted_iota(jnp.int32, (G, RB), 0) == g[None, :]
              ).astype(jnp.float32)
    part = jnp.dot(onehot, ls_ref[...], preferred_element_type=jnp.float32,
                   precision=lax.Precision.HIGHEST)

    @pl.when(i == 0)
    def _():
        acc_ref[...] = part

    @pl.when(i > 0)
    def _():
        acc_ref[...] += part

    @pl.when(i == pl.num_programs(0) - 1)
    def _():
        h1 = _selu(jnp.dot(acc_ref[...], w1_ref[...],
                           preferred_element_type=jnp.float32) + b1_ref[...])
        h2 = _selu(jnp.dot(h1, w2_ref[...],
                           preferred_element_type=jnp.float32) + b2_ref[...])
        out_ref[...] = jnp.dot(h2, w3_ref[...],
                               preferred_element_type=jnp.float32) + b3_ref[...]


def _tc_readout(gid3, ls, w1, b1, w2, b2, w3, b3):
    ru = w1.shape[1]
    return pl.pallas_call(
        _tc_readout_body,
        grid=(N // RB,),
        in_specs=[
            pl.BlockSpec((1, 1, RB), lambda i: (i, 0, 0)),
            pl.BlockSpec((RB, D), lambda i: (i, 0)),
            pl.BlockSpec((D, ru), lambda i: (0, 0)),
            pl.BlockSpec((1, ru), lambda i: (0, 0)),
            pl.BlockSpec((ru, ru), lambda i: (0, 0)),
            pl.BlockSpec((1, ru), lambda i: (0, 0)),
            pl.BlockSpec((ru, 1), lambda i: (0, 0)),
            pl.BlockSpec((1, 1), lambda i: (0, 0)),
        ],
        out_specs=pl.BlockSpec((G, 1), lambda i: (0, 0)),
        out_shape=jax.ShapeDtypeStruct((G, 1), jnp.float32),
        scratch_shapes=[pltpu.VMEM((G, D), jnp.float32)],
    )(gid3, ls, w1, b1, w2, b2, w3, b3)


def kernel(states_action, states_graph_ids, states_first, states_second,
           sates_num_edges, Wm, bm, Wx, Wh, b_gru,
           Wr1, br1, Wr2, br2, Wr3, br3):
    del sates_num_edges
    ls = states_action
    bm2 = bm.reshape(1, D)
    a, b = _tc_prep(ls, Wm, bm2)
    sc_edge = _get_sc_edge()
    for _ in range(T):
        parts = sc_edge(a, b, states_first, states_second)
        ls, a, b = _tc_gru(parts, ls, Wx, Wh, b_gru, Wm, bm2)
    gid3 = states_graph_ids.reshape(N // RB, 1, RB)
    return _tc_readout(gid3, ls, Wr1, br1.reshape(1, -1), Wr2,
                       br2.reshape(1, -1), Wr3, br3.reshape(1, 1))


# speedup vs baseline: 8.5597x; 1.7388x over previous
import functools

import jax
import jax.numpy as jnp
from jax import lax
from jax.experimental import pallas as pl
from jax.experimental.pallas import tpu as pltpu
from jax.experimental.pallas import tpu_sc as plsc

N = 10000
D = 128
E = 320000
G = 64
T = 4
RB = 2000
K = 80
NT = 32
EPT = E // NT
NCH = EPT // K
ACC_N = 10240
SUB_ROWS = ACC_N // 16

_SCALE = 1.0507009873554805
_ALPHA = 1.6732632423543772
_SA = _SCALE * _ALPHA


def _selu(x):
    return jnp.where(x > 0, _SCALE * x, _SA * jnp.exp(x) - _SA)


def _sc_edge_body(a_hbm, b_hbm, eidx_hbm, out_hbm,
                  acc_sh, ib_0, ib_1, ra_0, ra_1, rb_0, rb_1,
                  si_0, si_1, sa_0, sa_1, sb_0, sb_1):
    c = lax.axis_index("c")
    s = lax.axis_index("s")
    tile = c * 16 + s
    ib = (ib_0, ib_1)
    ra = (ra_0, ra_1)
    rb = (rb_0, rb_1)
    si = (si_0, si_1)
    sa = (sa_0, sa_1)
    sb = (sb_0, sb_1)
    zero = jnp.zeros((16,), jnp.float32)

    def ipre(chunk, q):
        pltpu.async_copy(eidx_hbm.at[tile].at[chunk], ib[q], si[q])

    def iwait(chunk, q):
        pltpu.make_async_copy(eidx_hbm.at[tile].at[chunk], ib[q], si[q]).wait()

    ipre(0, 0)
    ipre(1, 1)

    @pl.loop(0, K)
    def _zero_stage(r):
        for k in range(D // 16):
            ra_0[r, pl.ds(k * 16, 16)] = zero

    for b in range(SUB_ROWS // K):
        pltpu.sync_copy(ra_0, acc_sh.at[pl.ds(s * SUB_ROWS + b * K, K)])
    plsc.subcore_barrier()

    def start(chunk, p):
        iwait(chunk, p)
        pltpu.async_copy(a_hbm.at[ib[p].at[0]], ra[p], sa[p])
        pltpu.async_copy(b_hbm.at[ib[p].at[1]], rb[p], sb[p])

    def process(chunk, p):
        pltpu.make_async_copy(a_hbm.at[ib[p].at[0]], ra[p], sa[p]).wait()
        pltpu.make_async_copy(b_hbm.at[ib[p].at[1]], rb[p], sb[p]).wait()

        @plsc.parallel_loop(0, K)
        def _row(r):
            for k in range(D // 16):
                x = ra[p][r, pl.ds(k * 16, 16)] + rb[p][r, pl.ds(k * 16, 16)]
                ra[p][r, pl.ds(k * 16, 16)] = _selu(x)

        pltpu.sync_copy(ra[p], acc_sh.at[ib[p].at[1]], add=True)

        @pl.when(chunk + 2 < NCH)
        def _():
            ipre(chunk + 2, p)

    start(0, 0)

    @pl.loop(0, NCH - 1, step=2)
    def _pair(j):
        start(j + 1, 1)
        process(j, 0)
        start(j + 2, 0)
        process(j + 1, 1)

    process(NCH - 1, 0)

    plsc.subcore_barrier()
    for b in range(SUB_ROWS // K):
        row0 = s * SUB_ROWS + b * K
        pltpu.sync_copy(acc_sh.at[pl.ds(row0, K)], ra_0)
        pltpu.sync_copy(ra_0, out_hbm.at[c].at[pl.ds(row0, K)])


@functools.cache
def _get_sc_edge():
    mesh = plsc.VectorSubcoreMesh(core_axis_name="c", subcore_axis_name="s")
    return pl.kernel(
        _sc_edge_body,
        out_type=jax.ShapeDtypeStruct((2, ACC_N, D), jnp.float32),
        mesh=mesh,
        scratch_types=[
            pltpu.VMEM_SHARED((ACC_N, D), jnp.float32),
            pltpu.VMEM((2, K), jnp.int32),
            pltpu.VMEM((2, K), jnp.int32),
            pltpu.VMEM((K, D), jnp.float32),
            pltpu.VMEM((K, D), jnp.float32),
            pltpu.VMEM((K, D), jnp.float32),
            pltpu.VMEM((K, D), jnp.float32),
            pltpu.SemaphoreType.DMA,
            pltpu.SemaphoreType.DMA,
            pltpu.SemaphoreType.DMA,
            pltpu.SemaphoreType.DMA,
            pltpu.SemaphoreType.DMA,
            pltpu.SemaphoreType.DMA,
        ],
    )


def _tc_prep_body(ls_ref, wm_ref, bm_ref, a_ref, b_ref):
    x = ls_ref[...]
    wm = wm_ref[...]
    a_ref[...] = jnp.dot(x, wm[:D, :], preferred_element_type=jnp.float32)
    b_ref[...] = jnp.dot(x, wm[D:, :], preferred_element_type=jnp.float32) + bm_ref[...]


def _tc_prep(ls, wm, bm2):
    return pl.pallas_call(
        _tc_prep_body,
        grid=(N // RB,),
        in_specs=[
            pl.BlockSpec((RB, D), lambda i: (i, 0)),
            pl.BlockSpec((2 * D, D), lambda i: (0, 0)),
            pl.BlockSpec((1, D), lambda i: (0, 0)),
        ],
        out_specs=[pl.BlockSpec((RB, D), lambda i: (i, 0))] * 2,
        out_shape=[jax.ShapeDtypeStruct((N, D), jnp.float32)] * 2,
    )(ls, wm, bm2)


def _tc_gru_body(p_ref, ls_ref, wx_ref, wh_ref, bg_ref, wm_ref, bm_ref,
                 out_ls, out_a, out_b):
    x = p_ref[0] + p_ref[1]
    h = ls_ref[...]
    mx = jnp.dot(x, wx_ref[...], preferred_element_type=jnp.float32) + bg_ref[0:1, :]
    mh = jnp.dot(h, wh_ref[...], preferred_element_type=jnp.float32) + bg_ref[1:2, :]
    z = jax.nn.sigmoid(mx[:, :D] + mh[:, :D])
    r = jax.nn.sigmoid(mx[:, D:2 * D] + mh[:, D:2 * D])
    hh = jnp.tanh(mx[:, 2 * D:] + r * mh[:, 2 * D:])
    ls_new = z * h + (1.0 - z) * hh
    out_ls[...] = ls_new
    wm = wm_ref[...]
    out_a[...] = jnp.dot(ls_new, wm[:D, :], preferred_element_type=jnp.float32)
    out_b[...] = jnp.dot(ls_new, wm[D:, :], preferred_element_type=jnp.float32) + bm_ref[...]


def _tc_gru(parts, ls, wx, wh, bg, wm, bm2):
    return pl.pallas_call(
        _tc_gru_body,
        grid=(N // RB,),
        in_specs=[
            pl.BlockSpec((2, RB, D), lambda i: (0, i, 0)),
            pl.BlockSpec((RB, D), lambda i: (i, 0)),
            pl.BlockSpec((D, 3 * D), lambda i: (0, 0)),
            pl.BlockSpec((D, 3 * D), lambda i: (0, 0)),
            pl.BlockSpec((2, 3 * D), lambda i: (0, 0)),
            pl.BlockSpec((2 * D, D), lambda i: (0, 0)),
            pl.BlockSpec((1, D), lambda i: (0, 0)),
        ],
        out_specs=[pl.BlockSpec((RB, D), lambda i: (i, 0))] * 3,
        out_shape=[jax.ShapeDtypeStruct((N, D), jnp.float32)] * 3,
    )(parts, ls, wx, wh, bg, wm, bm2)


def _tc_readout_body(gid_ref, ls_ref, w1_ref, b1_ref, w2_ref, b2_ref,
                     w3_ref, b3_ref, out_ref, acc_ref):
    i = pl.program_id(0)
    g = gid_ref[0, 0, :]
    onehot = (lax.broadcasted_iota(jnp.int32, (G, RB), 0) == g[None, :]
              ).astype(jnp.float32)
    part = jnp.dot(onehot, ls_ref[...], preferred_element_type=jnp.float32,
                   precision=lax.Precision.HIGHEST)

    @pl.when(i == 0)
    def _():
        acc_ref[...] = part

    @pl.when(i > 0)
    def _():
        acc_ref[...] += part

    @pl.when(i == pl.num_programs(0) - 1)
    def _():
        h1 = _selu(jnp.dot(acc_ref[...], w1_ref[...],
                           preferred_element_type=jnp.float32) + b1_ref[...])
        h2 = _selu(jnp.dot(h1, w2_ref[...],
                           preferred_element_type=jnp.float32) + b2_ref[...])
        out_ref[...] = jnp.dot(h2, w3_ref[...],
                               preferred_element_type=jnp.float32) + b3_ref[...]


def _tc_readout(gid3, ls, w1, b1, w2, b2, w3, b3):
    ru = w1.shape[1]
    return pl.pallas_call(
        _tc_readout_body,
        grid=(N // RB,),
        in_specs=[
            pl.BlockSpec((1, 1, RB), lambda i: (i, 0, 0)),
            pl.BlockSpec((RB, D), lambda i: (i, 0)),
            pl.BlockSpec((D, ru), lambda i: (0, 0)),
            pl.BlockSpec((1, ru), lambda i: (0, 0)),
            pl.BlockSpec((ru, ru), lambda i: (0, 0)),
            pl.BlockSpec((1, ru), lambda i: (0, 0)),
            pl.BlockSpec((ru, 1), lambda i: (0, 0)),
            pl.BlockSpec((1, 1), lambda i: (0, 0)),
        ],
        out_specs=pl.BlockSpec((G, 1), lambda i: (0, 0)),
        out_shape=jax.ShapeDtypeStruct((G, 1), jnp.float32),
        scratch_shapes=[pltpu.VMEM((G, D), jnp.float32)],
    )(gid3, ls, w1, b1, w2, b2, w3, b3)


def kernel(states_action, states_graph_ids, states_first, states_second,
           sates_num_edges, Wm, bm, Wx, Wh, b_gru,
           Wr1, br1, Wr2, br2, Wr3, br3):
    del sates_num_edges
    ls = states_action
    bm2 = bm.reshape(1, D)
    a, b = _tc_prep(ls, Wm, bm2)
    sc_edge = _get_sc_edge()
    eidx = jnp.stack([states_first.reshape(NT, NCH, K),
                      states_second.reshape(NT, NCH, K)], axis=2)
    for _ in range(T):
        parts = sc_edge(a, b, eidx)
        ls, a, b = _tc_gru(parts, ls, Wx, Wh, b_gru, Wm, bm2)
    gid3 = states_graph_ids.reshape(N // RB, 1, RB)
    return _tc_readout(gid3, ls, Wr1, br1.reshape(1, -1), Wr2,
                       br2.reshape(1, -1), Wr3, br3.reshape(1, 1))


# speedup vs baseline: 10.2622x; 1.1989x over previous
import functools

import jax
import jax.numpy as jnp
from jax import lax
from jax.experimental import pallas as pl
from jax.experimental.pallas import tpu as pltpu
from jax.experimental.pallas import tpu_sc as plsc

N = 10000
D = 128
E = 320000
G = 64
T = 4
RB = 2000
K = 80
NT = 32
EPT = E // NT
NCH = EPT // K
ACC_N = 10240
SUB_ROWS = ACC_N // 16

_SCALE = 1.0507009873554805
_ALPHA = 1.6732632423543772
_SA = _SCALE * _ALPHA


def _selu(x):
    return jnp.where(x > 0, _SCALE * x, _SA * jnp.exp(x) - _SA)


def _sc_edge_body(a_hbm, b_hbm, eidx_hbm, out_hbm,
                  acc_sh, ib_0, ib_1, ib_2, ib_3, ra_0, ra_1, rb_0, rb_1,
                  si_0, si_1, si_2, si_3, sa_0, sa_1, sb_0, sb_1):
    c = lax.axis_index("c")
    s = lax.axis_index("s")
    tile = c * 16 + s
    ib = (ib_0, ib_1, ib_2, ib_3)
    ra = (ra_0, ra_1)
    rb = (rb_0, rb_1)
    si = (si_0, si_1, si_2, si_3)
    sa = (sa_0, sa_1)
    sb = (sb_0, sb_1)
    zero = jnp.zeros((16,), jnp.float32)

    def ipre(chunk, q):
        @pl.when(chunk < NCH)
        def _():
            pltpu.async_copy(eidx_hbm.at[tile].at[chunk], ib[q], si[q])

    def iwait(chunk, q):
        pltpu.make_async_copy(eidx_hbm.at[tile].at[chunk], ib[q], si[q]).wait()

    ipre(0, 0)
    ipre(1, 1)

    @pl.loop(0, K)
    def _zero_stage(r):
        for k in range(D // 16):
            ra_0[r, pl.ds(k * 16, 16)] = zero

    for b in range(SUB_ROWS // K):
        pltpu.sync_copy(ra_0, acc_sh.at[pl.ds(s * SUB_ROWS + b * K, K)])
    plsc.subcore_barrier()

    def start(chunk, p, q):
        iwait(chunk, q)
        pltpu.async_copy(a_hbm.at[ib[q].at[0]], ra[p], sa[p])
        pltpu.async_copy(b_hbm.at[ib[q].at[1]], rb[p], sb[p])
        ipre(chunk + 2, (q + 2) % 4)

    def process(chunk, p, q):
        pltpu.make_async_copy(a_hbm.at[ib[q].at[0]], ra[p], sa[p]).wait()
        pltpu.make_async_copy(b_hbm.at[ib[q].at[1]], rb[p], sb[p]).wait()

        @plsc.parallel_loop(0, K, unroll=2)
        def _row(r):
            for k in range(D // 16):
                x = ra[p][r, pl.ds(k * 16, 16)] + rb[p][r, pl.ds(k * 16, 16)]
                ra[p][r, pl.ds(k * 16, 16)] = jnp.minimum(
                    _SCALE * x, _SA * jnp.exp(x) - _SA)

        pltpu.sync_copy(ra[p], acc_sh.at[ib[q].at[1]], add=True)

    start(0, 0, 0)

    @pl.loop(0, NCH - 1, step=4)
    def _quad(j):
        start(j + 1, 1, 1)
        process(j, 0, 0)
        start(j + 2, 0, 2)
        process(j + 1, 1, 1)
        start(j + 3, 1, 3)
        process(j + 2, 0, 2)
        start(j + 4, 0, 0)
        process(j + 3, 1, 3)

    process(NCH - 1, 0, 0)

    plsc.subcore_barrier()
    for b in range(SUB_ROWS // K):
        row0 = s * SUB_ROWS + b * K
        pltpu.sync_copy(acc_sh.at[pl.ds(row0, K)], ra_0)
        pltpu.sync_copy(ra_0, out_hbm.at[c].at[pl.ds(row0, K)])


@functools.cache
def _get_sc_edge():
    mesh = plsc.VectorSubcoreMesh(core_axis_name="c", subcore_axis_name="s")
    return pl.kernel(
        _sc_edge_body,
        out_type=jax.ShapeDtypeStruct((2, ACC_N, D), jnp.float32),
        mesh=mesh,
        scratch_types=[
            pltpu.VMEM_SHARED((ACC_N, D), jnp.float32),
            pltpu.VMEM((2, K), jnp.int32),
            pltpu.VMEM((2, K), jnp.int32),
            pltpu.VMEM((2, K), jnp.int32),
            pltpu.VMEM((2, K), jnp.int32),
            pltpu.VMEM((K, D), jnp.float32),
            pltpu.VMEM((K, D), jnp.float32),
            pltpu.VMEM((K, D), jnp.float32),
            pltpu.VMEM((K, D), jnp.float32),
            pltpu.SemaphoreType.DMA,
            pltpu.SemaphoreType.DMA,
            pltpu.SemaphoreType.DMA,
            pltpu.SemaphoreType.DMA,
            pltpu.SemaphoreType.DMA,
            pltpu.SemaphoreType.DMA,
            pltpu.SemaphoreType.DMA,
            pltpu.SemaphoreType.DMA,
        ],
    )


def _tc_prep_body(ls_ref, wm_ref, bm_ref, a_ref, b_ref):
    x = ls_ref[...]
    wm = wm_ref[...]
    a_ref[...] = jnp.dot(x, wm[:D, :], preferred_element_type=jnp.float32)
    b_ref[...] = jnp.dot(x, wm[D:, :], preferred_element_type=jnp.float32) + bm_ref[...]


def _tc_prep(ls, wm, bm2):
    return pl.pallas_call(
        _tc_prep_body,
        grid=(N // RB,),
        in_specs=[
            pl.BlockSpec((RB, D), lambda i: (i, 0)),
            pl.BlockSpec((2 * D, D), lambda i: (0, 0)),
            pl.BlockSpec((1, D), lambda i: (0, 0)),
        ],
        out_specs=[pl.BlockSpec((RB, D), lambda i: (i, 0))] * 2,
        out_shape=[jax.ShapeDtypeStruct((N, D), jnp.float32)] * 2,
    )(ls, wm, bm2)


def _tc_gru_body(p_ref, ls_ref, wx_ref, wh_ref, bg_ref, wm_ref, bm_ref,
                 out_ls, out_a, out_b):
    x = p_ref[0] + p_ref[1]
    h = ls_ref[...]
    mx = jnp.dot(x, wx_ref[...], preferred_element_type=jnp.float32) + bg_ref[0:1, :]
    mh = jnp.dot(h, wh_ref[...], preferred_element_type=jnp.float32) + bg_ref[1:2, :]
    z = jax.nn.sigmoid(mx[:, :D] + mh[:, :D])
    r = jax.nn.sigmoid(mx[:, D:2 * D] + mh[:, D:2 * D])
    hh = jnp.tanh(mx[:, 2 * D:] + r * mh[:, 2 * D:])
    ls_new = z * h + (1.0 - z) * hh
    out_ls[...] = ls_new
    wm = wm_ref[...]
    out_a[...] = jnp.dot(ls_new, wm[:D, :], preferred_element_type=jnp.float32)
    out_b[...] = jnp.dot(ls_new, wm[D:, :], preferred_element_type=jnp.float32) + bm_ref[...]


def _tc_gru(parts, ls, wx, wh, bg, wm, bm2):
    return pl.pallas_call(
        _tc_gru_body,
        grid=(N // RB,),
        in_specs=[
            pl.BlockSpec((2, RB, D), lambda i: (0, i, 0)),
            pl.BlockSpec((RB, D), lambda i: (i, 0)),
            pl.BlockSpec((D, 3 * D), lambda i: (0, 0)),
            pl.BlockSpec((D, 3 * D), lambda i: (0, 0)),
            pl.BlockSpec((2, 3 * D), lambda i: (0, 0)),
            pl.BlockSpec((2 * D, D), lambda i: (0, 0)),
            pl.BlockSpec((1, D), lambda i: (0, 0)),
        ],
        out_specs=[pl.BlockSpec((RB, D), lambda i: (i, 0))] * 3,
        out_shape=[jax.ShapeDtypeStruct((N, D), jnp.float32)] * 3,
    )(parts, ls, wx, wh, bg, wm, bm2)


def _tc_readout_body(gid_ref, ls_ref, w1_ref, b1_ref, w2_ref, b2_ref,
                     w3_ref, b3_ref, out_ref, acc_ref):
    i = pl.program_id(0)
    g = gid_ref[0, 0, :]
    onehot = (lax.broadcasted_iota(jnp.int32, (G, RB), 0) == g[None, :]
              ).astype(jnp.float32)
    part = jnp.dot(onehot, ls_ref[...], preferred_element_type=jnp.float32,
                   precision=lax.Precision.HIGHEST)

    @pl.when(i == 0)
    def _():
        acc_ref[...] = part

    @pl.when(i > 0)
    def _():
        acc_ref[...] += part

    @pl.when(i == pl.num_programs(0) - 1)
    def _():
        h1 = _selu(jnp.dot(acc_ref[...], w1_ref[...],
                           preferred_element_type=jnp.float32) + b1_ref[...])
        h2 = _selu(jnp.dot(h1, w2_ref[...],
                           preferred_element_type=jnp.float32) + b2_ref[...])
        out_ref[...] = jnp.dot(h2, w3_ref[...],
                               preferred_element_type=jnp.float32) + b3_ref[...]


def _tc_readout(gid3, ls, w1, b1, w2, b2, w3, b3):
    ru = w1.shape[1]
    return pl.pallas_call(
        _tc_readout_body,
        grid=(N // RB,),
        in_specs=[
            pl.BlockSpec((1, 1, RB), lambda i: (i, 0, 0)),
            pl.BlockSpec((RB, D), lambda i: (i, 0)),
            pl.BlockSpec((D, ru), lambda i: (0, 0)),
            pl.BlockSpec((1, ru), lambda i: (0, 0)),
            pl.BlockSpec((ru, ru), lambda i: (0, 0)),
            pl.BlockSpec((1, ru), lambda i: (0, 0)),
            pl.BlockSpec((ru, 1), lambda i: (0, 0)),
            pl.BlockSpec((1, 1), lambda i: (0, 0)),
        ],
        out_specs=pl.BlockSpec((G, 1), lambda i: (0, 0)),
        out_shape=jax.ShapeDtypeStruct((G, 1), jnp.float32),
        scratch_shapes=[pltpu.VMEM((G, D), jnp.float32)],
    )(gid3, ls, w1, b1, w2, b2, w3, b3)


def kernel(states_action, states_graph_ids, states_first, states_second,
           sates_num_edges, Wm, bm, Wx, Wh, b_gru,
           Wr1, br1, Wr2, br2, Wr3, br3):
    del sates_num_edges
    ls = states_action
    bm2 = bm.reshape(1, D)
    a, b = _tc_prep(ls, Wm, bm2)
    sc_edge = _get_sc_edge()
    eidx = jnp.stack([states_first.reshape(NT, NCH, K),
                      states_second.reshape(NT, NCH, K)], axis=2)
    for _ in range(T):
        parts = sc_edge(a, b, eidx)
        ls, a, b = _tc_gru(parts, ls, Wx, Wh, b_gru, Wm, bm2)
    gid3 = states_graph_ids.reshape(N // RB, 1, RB)
    return _tc_readout(gid3, ls, Wr1, br1.reshape(1, -1), Wr2,
                       br2.reshape(1, -1), Wr3, br3.reshape(1, 1))
